# baseline (speedup 1.0000x reference)
import functools

import jax
import jax.numpy as jnp
from jax.experimental import pallas as pl
from jax.experimental.pallas import tpu as pltpu

_VMEM = 48 * 1024 * 1024


def _ceil_to(x, m):
    return (x + m - 1) // m * m


def _gemm_body(do_relu, with_res, a_ref, w_ref, b_ref, *refs):
    if with_res:
        res_ref, out_ref = refs
    else:
        (out_ref,) = refs
    acc = jnp.dot(a_ref[...], w_ref[...], preferred_element_type=jnp.float32)
    acc = acc + b_ref[...]
    if with_res:
        acc = acc + res_ref[...].astype(jnp.float32)
    if do_relu:
        acc = jnp.maximum(acc, 0.0)
    out_ref[...] = acc.astype(out_ref.dtype)


def _gemm(a, w, b, relu, residual=None, out_dtype=jnp.bfloat16):
    M, K = a.shape
    Kp, Np = w.shape
    tn = min(Np, 512)
    for cand in (512, 256, 128):
        if Np % cand == 0 and Np // cand >= 2:
            tn = cand
            break
    tm = min(512, _ceil_to(M, 8))
    Mp = _ceil_to(M, tm)
    a = a.astype(jnp.bfloat16)
    if Mp != M or Kp != K:
        a = jnp.pad(a, ((0, Mp - M), (0, Kp - K)))
    in_specs = [pl.BlockSpec((tm, Kp), lambda i, j: (i, 0)),
                pl.BlockSpec((Kp, tn), lambda i, j: (0, j)),
                pl.BlockSpec((1, tn), lambda i, j: (0, j))]
    args = [a, w, b]
    with_res = residual is not None
    if with_res:
        r = residual.astype(jnp.bfloat16)
        if Mp != M:
            r = jnp.pad(r, ((0, Mp - M), (0, 0)))
        in_specs.append(pl.BlockSpec((tm, tn), lambda i, j: (i, j)))
        args.append(r)
    out = pl.pallas_call(
        functools.partial(_gemm_body, relu, with_res),
        out_shape=jax.ShapeDtypeStruct((Mp, Np), out_dtype),
        grid=(Mp // tm, Np // tn),
        in_specs=in_specs,
        out_specs=pl.BlockSpec((tm, tn), lambda i, j: (i, j)),
        compiler_params=pltpu.CompilerParams(
            dimension_semantics=("parallel", "parallel"),
            vmem_limit_bytes=_VMEM),
    )(*args)
    return out[:M] if Mp != M else out


def _conv1x1(x, w, b, stride, relu, residual=None):
    if stride != 1:
        x = x[:, ::stride, ::stride, :]
    N, H, W, C = x.shape
    Np = w.shape[1]
    r2 = None if residual is None else residual.reshape(N * H * W, Np)
    out = _gemm(x.reshape(N * H * W, C), w, b, relu, r2)
    return out.reshape(N, H, W, Np)


def _conv_im2col(x, w, b, kh, kw, stride, pad, relu):
    N, H, W, C = x.shape
    xp = jnp.pad(x, ((0, 0), (pad, pad), (pad, pad), (0, 0)))
    OH = (H + 2 * pad - kh) // stride + 1
    OW = (W + 2 * pad - kw) // stride + 1
    cols = [xp[:, i:i + stride * OH:stride, j:j + stride * OW:stride, :]
            for i in range(kh) for j in range(kw)]
    pat = jnp.concatenate(cols, axis=-1).reshape(N * OH * OW, kh * kw * C)
    out = _gemm(pat, w, b, relu)
    return out.reshape(N, OH, OW, w.shape[1])


def _c3_body(H, W, C, x_ref, w_ref, b_ref, out_ref):
    xv = x_ref[...]
    acc = None
    for ki in range(3):
        for kj in range(3):
            xs = jax.lax.slice(xv, (0, ki, kj, 0), (1, ki + H, kj + W, C))
            xs = xs.reshape(H * W, C)
            t = ki * 3 + kj
            part = jnp.dot(xs, w_ref[t * C:(t + 1) * C, :],
                           preferred_element_type=jnp.float32)
            acc = part if acc is None else acc + part
    acc = acc + b_ref[...]
    out_ref[...] = jnp.maximum(acc, 0.0).astype(out_ref.dtype)


def _conv3_direct(x, w, b):
    N, H, W, C = x.shape
    Np = w.shape[1]
    tn = Np if Np <= 512 else 512
    xp = jnp.pad(x, ((0, 0), (1, 1), (1, 1), (0, 0)))
    out = pl.pallas_call(
        functools.partial(_c3_body, H, W, C),
        out_shape=jax.ShapeDtypeStruct((N * H * W, Np), jnp.bfloat16),
        grid=(N, Np // tn),
        in_specs=[
            pl.BlockSpec((1, H + 2, W + 2, C), lambda n, j: (n, 0, 0, 0)),
            pl.BlockSpec((9 * C, tn), lambda n, j: (0, j)),
            pl.BlockSpec((1, tn), lambda n, j: (0, j)),
        ],
        out_specs=pl.BlockSpec((H * W, tn), lambda n, j: (n, j)),
        compiler_params=pltpu.CompilerParams(
            dimension_semantics=("parallel", "parallel"),
            vmem_limit_bytes=_VMEM),
    )(xp, w, b)
    return out.reshape(N, H, W, Np)


def _max9_body(*refs):
    out_ref = refs[-1]
    m = refs[0][...]
    for r in refs[1:-1]:
        m = jnp.maximum(m, r[...])
    out_ref[...] = m


def _maxpool_3x3_s2(x):
    N, H, W, C = x.shape
    OH = (H - 1) // 2 + 1
    OW = (W - 1) // 2 + 1
    xp = jnp.pad(x, ((0, 0), (1, 1), (1, 1), (0, 0)))
    taps = [xp[:, i:i + 2 * OH:2, j:j + 2 * OW:2, :].reshape(N * OH * OW, C)
            for i in range(3) for j in range(3)]
    M = N * OH * OW
    tm = 512
    Mp = _ceil_to(M, tm)
    if Mp != M:
        taps = [jnp.pad(t, ((0, Mp - M), (0, 0))) for t in taps]
    spec = pl.BlockSpec((tm, C), lambda i: (i, 0))
    out = pl.pallas_call(
        _max9_body,
        out_shape=jax.ShapeDtypeStruct((Mp, C), x.dtype),
        grid=(Mp // tm,),
        in_specs=[spec] * 9,
        out_specs=spec,
        compiler_params=pltpu.CompilerParams(
            dimension_semantics=("parallel",),
            vmem_limit_bytes=_VMEM),
    )(*taps)
    return out[:M].reshape(N, OH, OW, C)


def _head_body(x_ref, w_ref, b_ref, out_ref):
    feat = jnp.mean(x_ref[...].astype(jnp.float32), axis=1)
    y = jnp.dot(feat.astype(jnp.bfloat16), w_ref[...],
                preferred_element_type=jnp.float32)
    out_ref[...] = y + b_ref[...]


def _head(x, w, b):
    N, H, W, C = x.shape
    Np = w.shape[1]
    tn = 256
    nb = 8
    xr = x.reshape(N, H * W, C)
    out = pl.pallas_call(
        _head_body,
        out_shape=jax.ShapeDtypeStruct((N, Np), jnp.float32),
        grid=(N // nb, Np // tn),
        in_specs=[
            pl.BlockSpec((nb, H * W, C), lambda n, j: (n, 0, 0)),
            pl.BlockSpec((C, tn), lambda n, j: (0, j)),
            pl.BlockSpec((1, tn), lambda n, j: (0, j)),
        ],
        out_specs=pl.BlockSpec((nb, tn), lambda n, j: (n, j)),
        compiler_params=pltpu.CompilerParams(
            dimension_semantics=("parallel", "parallel"),
            vmem_limit_bytes=_VMEM),
    )(xr, w, b)
    return out


def _bottleneck(x, c1w, c1b, c2w, c2b, c3w, c3b, stride, dsw=None, dsb=None):
    identity = x
    h = _conv1x1(x, c1w, c1b, 1, True)
    if stride == 1 and h.shape[2] % 8 == 0 and h.shape[3] >= 256:
        h = _conv3_direct(h, c2w, c2b)
    else:
        h = _conv_im2col(h, c2w, c2b, 3, 3, stride, 1, True)
    if dsw is not None:
        identity = _conv1x1(x, dsw, dsb, stride, False)
    return _conv1x1(h, c3w, c3b, 1, True, residual=identity)


def kernel(x, stem_w, stem_b, s0_0_conv1_w, s0_0_conv1_b, s0_0_conv2_w, s0_0_conv2_b, s0_0_conv3_w, s0_0_conv3_b, s0_0_ds_w, s0_0_ds_b, s0_1_conv1_w, s0_1_conv1_b, s0_1_conv2_w, s0_1_conv2_b, s0_1_conv3_w, s0_1_conv3_b, s0_2_conv1_w, s0_2_conv1_b, s0_2_conv2_w, s0_2_conv2_b, s0_2_conv3_w, s0_2_conv3_b, s1_0_conv1_w, s1_0_conv1_b, s1_0_conv2_w, s1_0_conv2_b, s1_0_conv3_w, s1_0_conv3_b, s1_0_ds_w, s1_0_ds_b, s1_1_conv1_w, s1_1_conv1_b, s1_1_conv2_w, s1_1_conv2_b, s1_1_conv3_w, s1_1_conv3_b, s1_2_conv1_w, s1_2_conv1_b, s1_2_conv2_w, s1_2_conv2_b, s1_2_conv3_w, s1_2_conv3_b, s1_3_conv1_w, s1_3_conv1_b, s1_3_conv2_w, s1_3_conv2_b, s1_3_conv3_w, s1_3_conv3_b, s2_0_conv1_w, s2_0_conv1_b, s2_0_conv2_w, s2_0_conv2_b, s2_0_conv3_w, s2_0_conv3_b, s2_0_ds_w, s2_0_ds_b, s2_1_conv1_w, s2_1_conv1_b, s2_1_conv2_w, s2_1_conv2_b, s2_1_conv3_w, s2_1_conv3_b, s2_2_conv1_w, s2_2_conv1_b, s2_2_conv2_w, s2_2_conv2_b, s2_2_conv3_w, s2_2_conv3_b, s2_3_conv1_w, s2_3_conv1_b, s2_3_conv2_w, s2_3_conv2_b, s2_3_conv3_w, s2_3_conv3_b, s2_4_conv1_w, s2_4_conv1_b, s2_4_conv2_w, s2_4_conv2_b, s2_4_conv3_w, s2_4_conv3_b, s2_5_conv1_w, s2_5_conv1_b, s2_5_conv2_w, s2_5_conv2_b, s2_5_conv3_w, s2_5_conv3_b, s3_0_conv1_w, s3_0_conv1_b, s3_0_conv2_w, s3_0_conv2_b, s3_0_conv3_w, s3_0_conv3_b, s3_0_ds_w, s3_0_ds_b, s3_1_conv1_w, s3_1_conv1_b, s3_1_conv2_w, s3_1_conv2_b, s3_1_conv3_w, s3_1_conv3_b, s3_2_conv1_w, s3_2_conv1_b, s3_2_conv2_w, s3_2_conv2_b, s3_2_conv3_w, s3_2_conv3_b, fc_w, fc_b):
    y = jnp.transpose(x, (0, 2, 3, 1)).astype(jnp.bfloat16)
    y = _conv_im2col(y, stem_w, stem_b, 7, 7, 2, 3, True)
    y = _maxpool_3x3_s2(y)
    blocks = [
        (s0_0_conv1_w, s0_0_conv1_b, s0_0_conv2_w, s0_0_conv2_b, s0_0_conv3_w, s0_0_conv3_b, 1, s0_0_ds_w, s0_0_ds_b),
        (s0_1_conv1_w, s0_1_conv1_b, s0_1_conv2_w, s0_1_conv2_b, s0_1_conv3_w, s0_1_conv3_b, 1, None, None),
        (s0_2_conv1_w, s0_2_conv1_b, s0_2_conv2_w, s0_2_conv2_b, s0_2_conv3_w, s0_2_conv3_b, 1, None, None),
        (s1_0_conv1_w, s1_0_conv1_b, s1_0_conv2_w, s1_0_conv2_b, s1_0_conv3_w, s1_0_conv3_b, 2, s1_0_ds_w, s1_0_ds_b),
        (s1_1_conv1_w, s1_1_conv1_b, s1_1_conv2_w, s1_1_conv2_b, s1_1_conv3_w, s1_1_conv3_b, 1, None, None),
        (s1_2_conv1_w, s1_2_conv1_b, s1_2_conv2_w, s1_2_conv2_b, s1_2_conv3_w, s1_2_conv3_b, 1, None, None),
        (s1_3_conv1_w, s1_3_conv1_b, s1_3_conv2_w, s1_3_conv2_b, s1_3_conv3_w, s1_3_conv3_b, 1, None, None),
        (s2_0_conv1_w, s2_0_conv1_b, s2_0_conv2_w, s2_0_conv2_b, s2_0_conv3_w, s2_0_conv3_b, 2, s2_0_ds_w, s2_0_ds_b),
        (s2_1_conv1_w, s2_1_conv1_b, s2_1_conv2_w, s2_1_conv2_b, s2_1_conv3_w, s2_1_conv3_b, 1, None, None),
        (s2_2_conv1_w, s2_2_conv1_b, s2_2_conv2_w, s2_2_conv2_b, s2_2_conv3_w, s2_2_conv3_b, 1, None, None),
        (s2_3_conv1_w, s2_3_conv1_b, s2_3_conv2_w, s2_3_conv2_b, s2_3_conv3_w, s2_3_conv3_b, 1, None, None),
        (s2_4_conv1_w, s2_4_conv1_b, s2_4_conv2_w, s2_4_conv2_b, s2_4_conv3_w, s2_4_conv3_b, 1, None, None),
        (s2_5_conv1_w, s2_5_conv1_b, s2_5_conv2_w, s2_5_conv2_b, s2_5_conv3_w, s2_5_conv3_b, 1, None, None),
        (s3_0_conv1_w, s3_0_conv1_b, s3_0_conv2_w, s3_0_conv2_b, s3_0_conv3_w, s3_0_conv3_b, 2, s3_0_ds_w, s3_0_ds_b),
        (s3_1_conv1_w, s3_1_conv1_b, s3_1_conv2_w, s3_1_conv2_b, s3_1_conv3_w, s3_1_conv3_b, 1, None, None),
        (s3_2_conv1_w, s3_2_conv1_b, s3_2_conv2_w, s3_2_conv2_b, s3_2_conv3_w, s3_2_conv3_b, 1, None, None),
    ]
    for blk in blocks:
        y = _bottleneck(y, *blk)
    logits = _head(y, fc_w, fc_b)
    return logits[:, :1000]


# speedup vs baseline: 1.3485x; 1.3485x over previous
import functools

import jax
import jax.numpy as jnp
from jax.experimental import pallas as pl
from jax.experimental.pallas import tpu as pltpu

_VMEM = 48 * 1024 * 1024


def _ceil_to(x, m):
    return (x + m - 1) // m * m


def _gemm_body(do_relu, with_res, a_ref, w_ref, b_ref, *refs):
    if with_res:
        res_ref, out_ref = refs
    else:
        (out_ref,) = refs
    acc = jnp.dot(a_ref[...], w_ref[...], preferred_element_type=jnp.float32)
    acc = acc + b_ref[...]
    if with_res:
        acc = acc + res_ref[...].astype(jnp.float32)
    if do_relu:
        acc = jnp.maximum(acc, 0.0)
    out_ref[...] = acc.astype(out_ref.dtype)


def _gemm(a, w, b, relu, residual=None, out_dtype=jnp.bfloat16):
    M, K = a.shape
    Kp, Np = w.shape
    tn = min(Np, 512)
    for cand in (512, 256, 128):
        if Np % cand == 0 and Np // cand >= 2:
            tn = cand
            break
    tm = min(512, _ceil_to(M, 8))
    Mp = _ceil_to(M, tm)
    a = a.astype(jnp.bfloat16)
    if Mp != M or Kp != K:
        a = jnp.pad(a, ((0, Mp - M), (0, Kp - K)))
    in_specs = [pl.BlockSpec((tm, Kp), lambda i, j: (i, 0)),
                pl.BlockSpec((Kp, tn), lambda i, j: (0, j)),
                pl.BlockSpec((1, tn), lambda i, j: (0, j))]
    args = [a, w, b]
    with_res = residual is not None
    if with_res:
        r = residual.astype(jnp.bfloat16)
        if Mp != M:
            r = jnp.pad(r, ((0, Mp - M), (0, 0)))
        in_specs.append(pl.BlockSpec((tm, tn), lambda i, j: (i, j)))
        args.append(r)
    out = pl.pallas_call(
        functools.partial(_gemm_body, relu, with_res),
        out_shape=jax.ShapeDtypeStruct((Mp, Np), out_dtype),
        grid=(Mp // tm, Np // tn),
        in_specs=in_specs,
        out_specs=pl.BlockSpec((tm, tn), lambda i, j: (i, j)),
        compiler_params=pltpu.CompilerParams(
            dimension_semantics=("parallel", "parallel"),
            vmem_limit_bytes=_VMEM),
    )(*args)
    return out[:M] if Mp != M else out


def _conv1x1(x, w, b, stride, relu, residual=None):
    if stride != 1:
        x = x[:, ::stride, ::stride, :]
    N, H, W, C = x.shape
    Np = w.shape[1]
    r2 = None if residual is None else residual.reshape(N * H * W, Np)
    out = _gemm(x.reshape(N * H * W, C), w, b, relu, r2)
    return out.reshape(N, H, W, Np)


def _conv_im2col(x, w, b, kh, kw, stride, pad, relu):
    N, H, W, C = x.shape
    xp = jnp.pad(x, ((0, 0), (pad, pad), (pad, pad), (0, 0)))
    OH = (H + 2 * pad - kh) // stride + 1
    OW = (W + 2 * pad - kw) // stride + 1
    cols = [xp[:, i:i + stride * OH:stride, j:j + stride * OW:stride, :]
            for i in range(kh) for j in range(kw)]
    pat = jnp.concatenate(cols, axis=-1).reshape(N * OH * OW, kh * kw * C)
    out = _gemm(pat, w, b, relu)
    return out.reshape(N, OH, OW, w.shape[1])


def _c3_body(H, W, C, x_ref, w_ref, b_ref, out_ref):
    xv = x_ref[...]
    acc = None
    for ki in range(3):
        for kj in range(3):
            xs = jax.lax.slice(xv, (0, ki, kj, 0), (1, ki + H, kj + W, C))
            xs = xs.reshape(H * W, C)
            t = ki * 3 + kj
            part = jnp.dot(xs, w_ref[t * C:(t + 1) * C, :],
                           preferred_element_type=jnp.float32)
            acc = part if acc is None else acc + part
    acc = acc + b_ref[...]
    out_ref[...] = jnp.maximum(acc, 0.0).astype(out_ref.dtype)


def _conv3_direct(x, w, b):
    N, H, W, C = x.shape
    Np = w.shape[1]
    tn = Np if Np <= 512 else 512
    xp = jnp.pad(x, ((0, 0), (1, 1), (1, 1), (0, 0)))
    out = pl.pallas_call(
        functools.partial(_c3_body, H, W, C),
        out_shape=jax.ShapeDtypeStruct((N * H * W, Np), jnp.bfloat16),
        grid=(N, Np // tn),
        in_specs=[
            pl.BlockSpec((1, H + 2, W + 2, C), lambda n, j: (n, 0, 0, 0)),
            pl.BlockSpec((9 * C, tn), lambda n, j: (0, j)),
            pl.BlockSpec((1, tn), lambda n, j: (0, j)),
        ],
        out_specs=pl.BlockSpec((H * W, tn), lambda n, j: (n, j)),
        compiler_params=pltpu.CompilerParams(
            dimension_semantics=("parallel", "parallel"),
            vmem_limit_bytes=_VMEM),
    )(xp, w, b)
    return out.reshape(N, H, W, Np)


def _max9_body(*refs):
    out_ref = refs[-1]
    m = refs[0][...]
    for r in refs[1:-1]:
        m = jnp.maximum(m, r[...])
    out_ref[...] = m


def _maxtaps(taps):
    M, C = taps[0].shape
    tm = 512
    Mp = _ceil_to(M, tm)
    if Mp != M:
        taps = [jnp.pad(t, ((0, Mp - M), (0, 0))) for t in taps]
    spec = pl.BlockSpec((tm, C), lambda i: (i, 0))
    out = pl.pallas_call(
        _max9_body,
        out_shape=jax.ShapeDtypeStruct((Mp, C), taps[0].dtype),
        grid=(Mp // tm,),
        in_specs=[spec] * len(taps),
        out_specs=spec,
        compiler_params=pltpu.CompilerParams(
            dimension_semantics=("parallel",),
            vmem_limit_bytes=_VMEM),
    )(*taps)
    return out[:M]


def _maxpool_3x3_s2(x):
    N, H, W, C = x.shape
    OH = (H - 1) // 2 + 1
    OW = (W - 1) // 2 + 1
    xp = jnp.pad(x, ((0, 0), (0, 0), (1, 1), (0, 0)))
    taps = [xp[:, :, j:j + 2 * OW:2, :].reshape(N * H * OW, C)
            for j in range(3)]
    y = _maxtaps(taps).reshape(N, H, OW, C)
    yp = jnp.pad(y, ((0, 0), (1, 1), (0, 0), (0, 0)))
    taps = [yp[:, i:i + 2 * OH:2, :, :].reshape(N * OH * OW, C)
            for i in range(3)]
    return _maxtaps(taps).reshape(N, OH, OW, C)


def _head_body(x_ref, w_ref, b_ref, out_ref):
    feat = jnp.mean(x_ref[...].astype(jnp.float32), axis=1)
    y = jnp.dot(feat.astype(jnp.bfloat16), w_ref[...],
                preferred_element_type=jnp.float32)
    out_ref[...] = y + b_ref[...]


def _head(x, w, b):
    N, H, W, C = x.shape
    Np = w.shape[1]
    tn = 256
    nb = 8
    xr = x.reshape(N, H * W, C)
    out = pl.pallas_call(
        _head_body,
        out_shape=jax.ShapeDtypeStruct((N, Np), jnp.float32),
        grid=(N // nb, Np // tn),
        in_specs=[
            pl.BlockSpec((nb, H * W, C), lambda n, j: (n, 0, 0)),
            pl.BlockSpec((C, tn), lambda n, j: (0, j)),
            pl.BlockSpec((1, tn), lambda n, j: (0, j)),
        ],
        out_specs=pl.BlockSpec((nb, tn), lambda n, j: (n, j)),
        compiler_params=pltpu.CompilerParams(
            dimension_semantics=("parallel", "parallel"),
            vmem_limit_bytes=_VMEM),
    )(xr, w, b)
    return out


def _bottleneck(x, c1w, c1b, c2w, c2b, c3w, c3b, stride, dsw=None, dsb=None):
    identity = x
    h = _conv1x1(x, c1w, c1b, 1, True)
    if stride == 1 and h.shape[2] % 8 == 0:
        h = _conv3_direct(h, c2w, c2b)
    else:
        h = _conv_im2col(h, c2w, c2b, 3, 3, stride, 1, True)
    if dsw is not None:
        identity = _conv1x1(x, dsw, dsb, stride, False)
    return _conv1x1(h, c3w, c3b, 1, True, residual=identity)


def kernel(x, stem_w, stem_b, s0_0_conv1_w, s0_0_conv1_b, s0_0_conv2_w, s0_0_conv2_b, s0_0_conv3_w, s0_0_conv3_b, s0_0_ds_w, s0_0_ds_b, s0_1_conv1_w, s0_1_conv1_b, s0_1_conv2_w, s0_1_conv2_b, s0_1_conv3_w, s0_1_conv3_b, s0_2_conv1_w, s0_2_conv1_b, s0_2_conv2_w, s0_2_conv2_b, s0_2_conv3_w, s0_2_conv3_b, s1_0_conv1_w, s1_0_conv1_b, s1_0_conv2_w, s1_0_conv2_b, s1_0_conv3_w, s1_0_conv3_b, s1_0_ds_w, s1_0_ds_b, s1_1_conv1_w, s1_1_conv1_b, s1_1_conv2_w, s1_1_conv2_b, s1_1_conv3_w, s1_1_conv3_b, s1_2_conv1_w, s1_2_conv1_b, s1_2_conv2_w, s1_2_conv2_b, s1_2_conv3_w, s1_2_conv3_b, s1_3_conv1_w, s1_3_conv1_b, s1_3_conv2_w, s1_3_conv2_b, s1_3_conv3_w, s1_3_conv3_b, s2_0_conv1_w, s2_0_conv1_b, s2_0_conv2_w, s2_0_conv2_b, s2_0_conv3_w, s2_0_conv3_b, s2_0_ds_w, s2_0_ds_b, s2_1_conv1_w, s2_1_conv1_b, s2_1_conv2_w, s2_1_conv2_b, s2_1_conv3_w, s2_1_conv3_b, s2_2_conv1_w, s2_2_conv1_b, s2_2_conv2_w, s2_2_conv2_b, s2_2_conv3_w, s2_2_conv3_b, s2_3_conv1_w, s2_3_conv1_b, s2_3_conv2_w, s2_3_conv2_b, s2_3_conv3_w, s2_3_conv3_b, s2_4_conv1_w, s2_4_conv1_b, s2_4_conv2_w, s2_4_conv2_b, s2_4_conv3_w, s2_4_conv3_b, s2_5_conv1_w, s2_5_conv1_b, s2_5_conv2_w, s2_5_conv2_b, s2_5_conv3_w, s2_5_conv3_b, s3_0_conv1_w, s3_0_conv1_b, s3_0_conv2_w, s3_0_conv2_b, s3_0_conv3_w, s3_0_conv3_b, s3_0_ds_w, s3_0_ds_b, s3_1_conv1_w, s3_1_conv1_b, s3_1_conv2_w, s3_1_conv2_b, s3_1_conv3_w, s3_1_conv3_b, s3_2_conv1_w, s3_2_conv1_b, s3_2_conv2_w, s3_2_conv2_b, s3_2_conv3_w, s3_2_conv3_b, fc_w, fc_b):
    y = jnp.transpose(x, (0, 2, 3, 1)).astype(jnp.bfloat16)
    y = _conv_im2col(y, stem_w, stem_b, 7, 7, 2, 3, True)
    y = _maxpool_3x3_s2(y)
    blocks = [
        (s0_0_conv1_w, s0_0_conv1_b, s0_0_conv2_w, s0_0_conv2_b, s0_0_conv3_w, s0_0_conv3_b, 1, s0_0_ds_w, s0_0_ds_b),
        (s0_1_conv1_w, s0_1_conv1_b, s0_1_conv2_w, s0_1_conv2_b, s0_1_conv3_w, s0_1_conv3_b, 1, None, None),
        (s0_2_conv1_w, s0_2_conv1_b, s0_2_conv2_w, s0_2_conv2_b, s0_2_conv3_w, s0_2_conv3_b, 1, None, None),
        (s1_0_conv1_w, s1_0_conv1_b, s1_0_conv2_w, s1_0_conv2_b, s1_0_conv3_w, s1_0_conv3_b, 2, s1_0_ds_w, s1_0_ds_b),
        (s1_1_conv1_w, s1_1_conv1_b, s1_1_conv2_w, s1_1_conv2_b, s1_1_conv3_w, s1_1_conv3_b, 1, None, None),
        (s1_2_conv1_w, s1_2_conv1_b, s1_2_conv2_w, s1_2_conv2_b, s1_2_conv3_w, s1_2_conv3_b, 1, None, None),
        (s1_3_conv1_w, s1_3_conv1_b, s1_3_conv2_w, s1_3_conv2_b, s1_3_conv3_w, s1_3_conv3_b, 1, None, None),
        (s2_0_conv1_w, s2_0_conv1_b, s2_0_conv2_w, s2_0_conv2_b, s2_0_conv3_w, s2_0_conv3_b, 2, s2_0_ds_w, s2_0_ds_b),
        (s2_1_conv1_w, s2_1_conv1_b, s2_1_conv2_w, s2_1_conv2_b, s2_1_conv3_w, s2_1_conv3_b, 1, None, None),
        (s2_2_conv1_w, s2_2_conv1_b, s2_2_conv2_w, s2_2_conv2_b, s2_2_conv3_w, s2_2_conv3_b, 1, None, None),
        (s2_3_conv1_w, s2_3_conv1_b, s2_3_conv2_w, s2_3_conv2_b, s2_3_conv3_w, s2_3_conv3_b, 1, None, None),
        (s2_4_conv1_w, s2_4_conv1_b, s2_4_conv2_w, s2_4_conv2_b, s2_4_conv3_w, s2_4_conv3_b, 1, None, None),
        (s2_5_conv1_w, s2_5_conv1_b, s2_5_conv2_w, s2_5_conv2_b, s2_5_conv3_w, s2_5_conv3_b, 1, None, None),
        (s3_0_conv1_w, s3_0_conv1_b, s3_0_conv2_w, s3_0_conv2_b, s3_0_conv3_w, s3_0_conv3_b, 2, s3_0_ds_w, s3_0_ds_b),
        (s3_1_conv1_w, s3_1_conv1_b, s3_1_conv2_w, s3_1_conv2_b, s3_1_conv3_w, s3_1_conv3_b, 1, None, None),
        (s3_2_conv1_w, s3_2_conv1_b, s3_2_conv2_w, s3_2_conv2_b, s3_2_conv3_w, s3_2_conv3_b, 1, None, None),
    ]
    for blk in blocks:
        y = _bottleneck(y, *blk)
    logits = _head(y, fc_w, fc_b)
    return logits[:, :1000]
